# Initial kernel scaffold; baseline (speedup 1.0000x reference)
import jax
import jax.numpy as jnp
from jax.experimental import pallas as pl


def kernel(x, allele_embedding, position_embedding):
    raise NotImplementedError("write your pallas kernel here")


# speedup vs baseline: 1.6595x; 1.6595x over previous
import functools

import jax
import jax.numpy as jnp
from jax.experimental import pallas as pl
from jax.experimental.pallas import tpu as pltpu

S_BLK = 512


def _geno_block(x_ref, a_ref, p_ref, o_ref):
    p = p_ref[...]
    a = a_ref[...]
    x = x_ref[...]
    b = x.shape[0]
    n = a.shape[0]
    for bi in range(b):
        acc = p
        for ni in range(n):
            acc = acc + x[bi, :, ni][:, None] * a[ni][None, :]
        o_ref[bi] = acc


@jax.jit
def kernel(x, allele_embedding, position_embedding):
    B, S, N = x.shape
    D = allele_embedding.shape[1]
    grid = (S // S_BLK,)
    out = pl.pallas_call(
        _geno_block,
        grid=grid,
        in_specs=[
            pl.BlockSpec((B, S_BLK, N), lambda i: (0, i, 0)),
            pl.BlockSpec((N, D), lambda i: (0, 0)),
            pl.BlockSpec((S_BLK, D), lambda i: (i, 0)),
        ],
        out_specs=pl.BlockSpec((B, S_BLK, D), lambda i: (0, i, 0)),
        out_shape=jax.ShapeDtypeStruct((B, S, D), jnp.float32),
    )(x, allele_embedding, position_embedding)
    return out


# speedup vs baseline: 1.7584x; 1.0596x over previous
import functools

import jax
import jax.numpy as jnp
from jax.experimental import pallas as pl
from jax.experimental.pallas import tpu as pltpu

S_BLK = 512


def _geno_block(x_ref, a_ref, p_ref, o_ref):
    p = p_ref[...]
    a = a_ref[...]
    x = x_ref[...]
    b = x.shape[0]
    for bi in range(b):
        y = jnp.dot(x[bi], a, preferred_element_type=jnp.float32)
        o_ref[bi] = y + p


@jax.jit
def kernel(x, allele_embedding, position_embedding):
    B, S, N = x.shape
    D = allele_embedding.shape[1]
    grid = (S // S_BLK,)
    out = pl.pallas_call(
        _geno_block,
        grid=grid,
        in_specs=[
            pl.BlockSpec((B, S_BLK, N), lambda i: (0, i, 0)),
            pl.BlockSpec((N, D), lambda i: (0, 0)),
            pl.BlockSpec((S_BLK, D), lambda i: (i, 0)),
        ],
        out_specs=pl.BlockSpec((B, S_BLK, D), lambda i: (0, i, 0)),
        out_shape=jax.ShapeDtypeStruct((B, S, D), jnp.float32),
    )(x, allele_embedding, position_embedding)
    return out


# speedup vs baseline: 1.8039x; 1.0259x over previous
import functools

import jax
import jax.numpy as jnp
from jax.experimental import pallas as pl
from jax.experimental.pallas import tpu as pltpu

S_BLK = 1024


def _geno_block(x_ref, a_ref, p_ref, o_ref):
    p = p_ref[...]
    a = a_ref[...]
    x = x_ref[...]
    b = x.shape[0]
    for bi in range(b):
        y = jnp.dot(x[bi], a, preferred_element_type=jnp.float32)
        o_ref[bi] = y + p


@jax.jit
def kernel(x, allele_embedding, position_embedding):
    B, S, N = x.shape
    D = allele_embedding.shape[1]
    grid = (S // S_BLK,)
    out = pl.pallas_call(
        _geno_block,
        grid=grid,
        in_specs=[
            pl.BlockSpec((B, S_BLK, N), lambda i: (0, i, 0)),
            pl.BlockSpec((N, D), lambda i: (0, 0)),
            pl.BlockSpec((S_BLK, D), lambda i: (i, 0)),
        ],
        out_specs=pl.BlockSpec((B, S_BLK, D), lambda i: (0, i, 0)),
        out_shape=jax.ShapeDtypeStruct((B, S, D), jnp.float32),
    )(x, allele_embedding, position_embedding)
    return out
